# Initial kernel scaffold; baseline (speedup 1.0000x reference)
import jax
import jax.numpy as jnp
from jax.experimental import pallas as pl


def kernel(x_cat, x_cont, tables, cont_mean, cont_std):
    raise NotImplementedError("write your pallas kernel here")


# speedup vs baseline: 8.8441x; 8.8441x over previous
import functools

import jax
import jax.numpy as jnp
from jax import lax
from jax.experimental import pallas as pl
from jax.experimental.pallas import tpu as pltpu
from jax.experimental.pallas import tpu_sc as plsc

N_FIELDS = 26
VOCAB = 100000
EMB = 32
N_CONT = 13
BATCH = 16384
OUT_W = N_FIELDS * EMB + N_CONT

NC = 2
NS = 16
NW = NC * NS
BPW = BATCH // NW
GATHER_CHUNK = 128
NCHUNK = BPW // GATHER_CHUNK


def _body(xcat_hbm, xcont_hbm, table_hbm, mean_hbm, std_hbm, out_hbm,
          xcat_v, xcont_v, idx_v, rows_v, cont_v, mean_v, std_v,
          sem_in, sem_g, sem_w):
    wid = lax.axis_index("s") * NC + lax.axis_index("c")
    base = wid * BPW

    cp_cat = pltpu.async_copy(xcat_hbm.at[pl.ds(base, BPW), :], xcat_v, sem_in)
    cp_cont = pltpu.async_copy(xcont_hbm.at[pl.ds(base, BPW), :], xcont_v, sem_in)
    cp_mean = pltpu.async_copy(mean_hbm, mean_v, sem_in)
    cp_std = pltpu.async_copy(std_hbm, std_v, sem_in)
    cp_cat.wait()
    cp_cont.wait()
    cp_mean.wait()
    cp_std.wait()

    lane = lax.iota(jnp.int32, 16)

    prev_write = None
    for f in range(N_FIELDS):
        col = jnp.full((16,), f, dtype=jnp.int32)
        off = f * VOCAB
        for k in range(NCHUNK):
            def build(i, _, k=k, col=col, off=off):
                row = lane + (k * GATHER_CHUNK + i * 16)
                v = plsc.load_gather(xcat_v, [row, col]) + off
                idx_v[k, pl.ds(i * 16, 16)] = v
                return _
            lax.fori_loop(0, GATHER_CHUNK // 16, build, 0)
        if prev_write is not None:
            prev_write.wait()
        gathers = []
        for k in range(NCHUNK):
            gathers.append(pltpu.async_copy(
                table_hbm.at[idx_v.at[k]],
                rows_v.at[pl.ds(k * GATHER_CHUNK, GATHER_CHUNK), :],
                sem_g))
        for cp in gathers:
            cp.wait()
        prev_write = pltpu.async_copy(
            rows_v, out_hbm.at[pl.ds(base, BPW), pl.ds(f * EMB, EMB)], sem_w)

    for c in range(N_CONT):
        colc = jnp.full((16,), c, dtype=jnp.int32)
        m = mean_v[c, :]
        s = std_v[c, :]
        def cont_body(i, _, colc=colc, m=m, s=s):
            row = lane + i * 16
            v = plsc.load_gather(xcont_v, [row, colc])
            plsc.store_scatter(cont_v, [row, colc], (v - m) / s)
            return _
        lax.fori_loop(0, BPW // 16, cont_body, 0)
    cp_c = pltpu.async_copy(
        cont_v, out_hbm.at[pl.ds(base, BPW), pl.ds(N_FIELDS * EMB, N_CONT)],
        sem_w)
    prev_write.wait()
    cp_c.wait()


@jax.jit
def _sc_call(xcat, xcont, flat_table, mean_b, std_b):
    kfn = pl.kernel(
        _body,
        out_type=jax.ShapeDtypeStruct((BATCH, OUT_W), jnp.float32),
        mesh=plsc.VectorSubcoreMesh(core_axis_name="c", subcore_axis_name="s"),
        scratch_types=[
            pltpu.VMEM((BPW, N_FIELDS), jnp.int32),
            pltpu.VMEM((BPW, N_CONT), jnp.float32),
            pltpu.VMEM((NCHUNK, GATHER_CHUNK), jnp.int32),
            pltpu.VMEM((BPW, EMB), jnp.float32),
            pltpu.VMEM((BPW, N_CONT), jnp.float32),
            pltpu.VMEM((N_CONT, 16), jnp.float32),
            pltpu.VMEM((N_CONT, 16), jnp.float32),
            pltpu.SemaphoreType.DMA,
            pltpu.SemaphoreType.DMA,
            pltpu.SemaphoreType.DMA,
        ],
        compiler_params=pltpu.CompilerParams(
            use_tc_tiling_on_sc=False, needs_layout_passes=False),
    )
    return kfn(xcat, xcont, flat_table, mean_b, std_b)


def kernel(x_cat, x_cont, tables, cont_mean, cont_std):
    flat_table = tables.reshape(N_FIELDS * VOCAB, EMB)
    mean_b = jnp.broadcast_to(cont_mean[:, None], (N_CONT, 16))
    std_b = jnp.broadcast_to(cont_std[:, None], (N_CONT, 16))
    return _sc_call(x_cat, x_cont, flat_table, mean_b, std_b)
